# baseline (speedup 1.0000x reference)
import functools

import jax
import jax.numpy as jnp
from jax import lax
from jax.experimental import pallas as pl
from jax.experimental.pallas import tpu as pltpu
from jax.experimental.pallas import tpu_sc as plsc

BATCH = 16384
DIM = 64
VOCAB = 1000000

_info = plsc.get_sparse_core_info()
_NC, _NS, _NL = _info.num_cores, _info.num_subcores, _info.num_lanes
_NW = _NC * _NS
_RANGE = 245 * 128
_FULL = (VOCAB // 128) * 128
_WIN = 256
_RING = 64

_mesh = plsc.VectorSubcoreMesh(core_axis_name="c", subcore_axis_name="s")


@functools.partial(
    pl.kernel,
    mesh=_mesh,
    out_type=jax.ShapeDtypeStruct((BATCH // 8, 8, DIM), jnp.float32),
    scratch_types=[
        pltpu.VMEM((BATCH,), jnp.int32),
        pltpu.VMEM((BATCH + _NL,), jnp.int32),
        pltpu.VMEM((BATCH + _NL,), jnp.int32),
        pltpu.VMEM((BATCH + _NL,), jnp.int32),
        pltpu.VMEM((2, DIM, _WIN), jnp.float32),
        pltpu.VMEM((DIM, DIM), jnp.float32),
        pltpu.VMEM((_RING, DIM), jnp.float32),
        pltpu.VMEM((_NL,), jnp.int32),
        pltpu.VMEM((_NL,), jnp.int32),
        pltpu.SemaphoreType.DMA,
        pltpu.SemaphoreType.DMA,
        pltpu.SemaphoreType.DMA,
    ],
    compiler_params=pltpu.CompilerParams(needs_layout_passes=False),
)
def _gather_kernel(idx_hbm, tab_hbm, tail_hbm, out_hbm, gidx_v, oidx_v,
                   opos_v, owin_v, win_v, tail_v, rows_v, tmpi_v, tmpp_v,
                   wsem0, wsem1, rsem):
    wid = lax.axis_index("s") * _NC + lax.axis_index("c")
    lo = wid * _RANGE
    hi_fb = jnp.minimum(lo + _RANGE, _FULL)
    nwin = (hi_fb - lo + _WIN - 1) // _WIN
    lane = lax.iota(jnp.int32, _NL)
    wsems = (wsem0, wsem1)

    pltpu.sync_copy(idx_hbm, gidx_v)
    pltpu.sync_copy(tail_hbm, tail_v)

    def scan_body(k, off):
        vec = gidx_v[pl.ds(k * _NL, _NL)]
        m = (vec // _RANGE) == wid
        plsc.store_compressed(oidx_v.at[pl.ds(off, _NL)], vec, mask=m)
        plsc.store_compressed(
            opos_v.at[pl.ds(off, _NL)], lane + k * _NL, mask=m
        )
        return off + jnp.sum(jnp.where(m, 1, 0))

    count = lax.fori_loop(0, BATCH // _NL, scan_body, 0)
    nvreg = (count + _NL - 1) // _NL

    def wtag_body(k, carry):
        vec = oidx_v[pl.ds(k * _NL, _NL)]
        wv = jnp.minimum((vec - lo) // _WIN, nwin - 1)
        wv = jnp.where(vec >= _FULL, nwin, wv)
        valid = (lane + k * _NL) < count
        owin_v[pl.ds(k * _NL, _NL)] = jnp.where(valid, wv, -1)
        return carry

    lax.fori_loop(0, nvreg, wtag_body, 0)

    def fire_win(w, slot):
        off = pl.multiple_of(jnp.minimum(lo + w * _WIN, hi_fb - _WIN), 128)
        pltpu.async_copy(
            tab_hbm.at[pl.ds(0, DIM), pl.ds(off, _WIN)],
            win_v.at[slot],
            wsems[slot],
        )

    def extract(kctr, pos_e, col_vec, src_ref):
        @pl.when(kctr >= _RING)
        def _():
            pltpu.make_async_copy(
                tail_hbm.at[0], rows_v.at[0], rsem
            ).wait()

        slot_r = kctr & (_RING - 1)
        for g in range(DIM // _NL):
            v = plsc.load_gather(src_ref, [lane + g * _NL, col_vec])
            rows_v[slot_r, pl.ds(g * _NL, _NL)] = v
        pltpu.async_copy(
            rows_v.at[slot_r],
            out_hbm.at[pos_e >> 3, pos_e & 7],
            rsem,
        )
        return kctr + 1

    def process_win(w, slot, kctr):
        off = jnp.minimum(lo + w * _WIN, hi_fb - _WIN)

        def vreg_body(k, kctr):
            ovec = oidx_v[pl.ds(k * _NL, _NL)]
            wv = owin_v[pl.ds(k * _NL, _NL)]
            m = wv == w
            cnt = jnp.sum(jnp.where(m, 1, 0))

            def entry_body(j, kctr):
                sel = lane == j
                idx_e = jnp.sum(jnp.where(sel, tmpi_v[...], 0))
                pos_e = jnp.sum(jnp.where(sel, tmpp_v[...], 0))
                col = jnp.broadcast_to(idx_e - off, (_NL,))
                return extract(kctr, pos_e, col, win_v.at[slot])

            @pl.when(cnt > 0)
            def _():
                plsc.store_compressed(tmpi_v.at[pl.ds(0, _NL)], ovec, mask=m)
                plsc.store_compressed(
                    tmpp_v.at[pl.ds(0, _NL)],
                    opos_v[pl.ds(k * _NL, _NL)],
                    mask=m,
                )

            return lax.fori_loop(0, cnt, entry_body, kctr)

        return lax.fori_loop(0, nvreg, vreg_body, kctr)

    fire_win(0, 0)

    def win_pair_body(g, kctr):
        for s in range(2):
            w = g * 2 + s

            def do(kctr=kctr, w=w, s=s):
                pltpu.make_async_copy(
                    tab_hbm.at[pl.ds(0, DIM), pl.ds(0, _WIN)],
                    win_v.at[s],
                    wsems[s],
                ).wait()

                @pl.when(w + 1 < nwin)
                def _():
                    fire_win(w + 1, 1 - s)

                return process_win(w, s, kctr)

            kctr = lax.cond(w < nwin, do, lambda kctr=kctr: kctr)
        return kctr

    kctr = lax.fori_loop(0, (nwin + 1) // 2, win_pair_body, 0)

    def tail_vreg_body(k, kctr):
        ovec = oidx_v[pl.ds(k * _NL, _NL)]
        wv = owin_v[pl.ds(k * _NL, _NL)]
        m = wv == nwin
        cnt = jnp.sum(jnp.where(m, 1, 0))

        def entry_body(j, kctr):
            sel = lane == j
            idx_e = jnp.sum(jnp.where(sel, tmpi_v[...], 0))
            pos_e = jnp.sum(jnp.where(sel, tmpp_v[...], 0))

            @pl.when(kctr >= _RING)
            def _():
                pltpu.make_async_copy(
                    tail_hbm.at[0], rows_v.at[0], rsem
                ).wait()

            slot_r = kctr & (_RING - 1)
            rv = jnp.broadcast_to(idx_e - _FULL, (_NL,))
            for g in range(DIM // _NL):
                rows_v[slot_r, pl.ds(g * _NL, _NL)] = plsc.load_gather(
                    tail_v, [rv, lane + g * _NL]
                )
            pltpu.async_copy(
                rows_v.at[slot_r],
                out_hbm.at[pos_e >> 3, pos_e & 7],
                rsem,
            )
            return kctr + 1

        @pl.when(cnt > 0)
        def _():
            plsc.store_compressed(tmpi_v.at[pl.ds(0, _NL)], ovec, mask=m)
            plsc.store_compressed(
                tmpp_v.at[pl.ds(0, _NL)], opos_v[pl.ds(k * _NL, _NL)], mask=m
            )

        return lax.fori_loop(0, cnt, entry_body, kctr)

    kctr = lax.fori_loop(0, nvreg, tail_vreg_body, kctr)

    def drain_body(d, carry):
        pltpu.make_async_copy(tail_hbm.at[0], rows_v.at[0], rsem).wait()
        return carry

    lax.fori_loop(0, jnp.minimum(kctr, _RING), drain_body, 0)


def kernel(sample_ids, codebook):
    idx = sample_ids.astype(jnp.int32)
    tail = codebook[_FULL:]
    out3 = _gather_kernel(idx, codebook.T, tail)
    return out3.reshape(BATCH, DIM)


# speedup vs baseline: 1.2119x; 1.2119x over previous
import functools

import jax
import jax.numpy as jnp
from jax import lax
from jax.experimental import pallas as pl
from jax.experimental.pallas import tpu as pltpu
from jax.experimental.pallas import tpu_sc as plsc

BATCH = 16384
DIM = 64
VOCAB = 1000000

_info = plsc.get_sparse_core_info()
_NC, _NS, _NL = _info.num_cores, _info.num_subcores, _info.num_lanes
_NW = _NC * _NS
_RANGE = 245 * 128
_FULL = (VOCAB // 128) * 128
_WIN = 512
_RING = 16

_mesh = plsc.VectorSubcoreMesh(core_axis_name="c", subcore_axis_name="s")


@functools.partial(
    pl.kernel,
    mesh=_mesh,
    out_type=jax.ShapeDtypeStruct((BATCH // 8, 8, DIM), jnp.float32),
    scratch_types=[
        pltpu.VMEM((4096,), jnp.int32),
        pltpu.VMEM((BATCH + _NL,), jnp.int32),
        pltpu.VMEM((BATCH + _NL,), jnp.int32),
        pltpu.VMEM((BATCH + _NL,), jnp.int32),
        pltpu.VMEM((2, DIM, _WIN), jnp.float32),
        pltpu.VMEM((DIM, DIM), jnp.float32),
        pltpu.VMEM((_RING, DIM), jnp.float32),
        pltpu.VMEM((_NL,), jnp.int32),
        pltpu.VMEM((_NL,), jnp.int32),
        pltpu.SemaphoreType.DMA,
        pltpu.SemaphoreType.DMA,
        pltpu.SemaphoreType.DMA,
    ],
    compiler_params=pltpu.CompilerParams(needs_layout_passes=False),
)
def _gather_kernel(idx_hbm, tab_hbm, tail_hbm, out_hbm, gidx_v, oidx_v,
                   opos_v, owin_v, win_v, tail_v, rows_v, tmpi_v, tmpp_v,
                   wsem0, wsem1, rsem):
    wid = lax.axis_index("s") * _NC + lax.axis_index("c")
    lo = wid * _RANGE
    hi_fb = jnp.minimum(lo + _RANGE, _FULL)
    nwin = (hi_fb - lo + _WIN - 1) // _WIN
    lane = lax.iota(jnp.int32, _NL)
    wsems = (wsem0, wsem1)

    pltpu.sync_copy(tail_hbm, tail_v)

    def chunk_body(c, off):
        pltpu.sync_copy(idx_hbm.at[pl.ds(c * 4096, 4096)], gidx_v)

        def scan_body(k, off):
            vec = gidx_v[pl.ds(k * _NL, _NL)]
            m = (vec // _RANGE) == wid
            plsc.store_compressed(oidx_v.at[pl.ds(off, _NL)], vec, mask=m)
            plsc.store_compressed(
                opos_v.at[pl.ds(off, _NL)],
                lane + (c * 4096 // _NL + k) * _NL,
                mask=m,
            )
            return off + jnp.sum(jnp.where(m, 1, 0))

        return lax.fori_loop(0, 4096 // _NL, scan_body, off)

    count = lax.fori_loop(0, BATCH // 4096, chunk_body, 0)
    nvreg = (count + _NL - 1) // _NL

    def wtag_body(k, carry):
        vec = oidx_v[pl.ds(k * _NL, _NL)]
        wv = jnp.minimum((vec - lo) // _WIN, nwin - 1)
        wv = jnp.where(vec >= _FULL, nwin, wv)
        valid = (lane + k * _NL) < count
        owin_v[pl.ds(k * _NL, _NL)] = jnp.where(valid, wv, -1)
        return carry

    lax.fori_loop(0, nvreg, wtag_body, 0)

    def fire_win(w, slot):
        off = pl.multiple_of(jnp.minimum(lo + w * _WIN, hi_fb - _WIN), 128)
        pltpu.async_copy(
            tab_hbm.at[pl.ds(0, DIM), pl.ds(off, _WIN)],
            win_v.at[slot],
            wsems[slot],
        )

    def extract(kctr, pos_e, col_vec, src_ref):
        @pl.when(kctr >= _RING)
        def _():
            pltpu.make_async_copy(
                tail_hbm.at[0], rows_v.at[0], rsem
            ).wait()

        slot_r = kctr & (_RING - 1)
        for g in range(DIM // _NL):
            v = plsc.load_gather(src_ref, [lane + g * _NL, col_vec])
            rows_v[slot_r, pl.ds(g * _NL, _NL)] = v
        pltpu.async_copy(
            rows_v.at[slot_r],
            out_hbm.at[pos_e >> 3, pos_e & 7],
            rsem,
        )
        return kctr + 1

    def process_win(w, slot, kctr):
        off = jnp.minimum(lo + w * _WIN, hi_fb - _WIN)

        def vreg_body(k, kctr):
            ovec = oidx_v[pl.ds(k * _NL, _NL)]
            wv = owin_v[pl.ds(k * _NL, _NL)]
            m = wv == w
            cnt = jnp.sum(jnp.where(m, 1, 0))

            def entry_body(j, kctr):
                sel = lane == j
                idx_e = jnp.sum(jnp.where(sel, tmpi_v[...], 0))
                pos_e = jnp.sum(jnp.where(sel, tmpp_v[...], 0))
                col = jnp.broadcast_to(idx_e - off, (_NL,))
                return extract(kctr, pos_e, col, win_v.at[slot])

            @pl.when(cnt > 0)
            def _():
                plsc.store_compressed(tmpi_v.at[pl.ds(0, _NL)], ovec, mask=m)
                plsc.store_compressed(
                    tmpp_v.at[pl.ds(0, _NL)],
                    opos_v[pl.ds(k * _NL, _NL)],
                    mask=m,
                )

            return lax.fori_loop(0, cnt, entry_body, kctr)

        return lax.fori_loop(0, nvreg, vreg_body, kctr)

    fire_win(0, 0)

    def win_pair_body(g, kctr):
        for s in range(2):
            w = g * 2 + s

            def do(kctr=kctr, w=w, s=s):
                pltpu.make_async_copy(
                    tab_hbm.at[pl.ds(0, DIM), pl.ds(0, _WIN)],
                    win_v.at[s],
                    wsems[s],
                ).wait()

                @pl.when(w + 1 < nwin)
                def _():
                    fire_win(w + 1, 1 - s)

                return process_win(w, s, kctr)

            kctr = lax.cond(w < nwin, do, lambda kctr=kctr: kctr)
        return kctr

    kctr = lax.fori_loop(0, (nwin + 1) // 2, win_pair_body, 0)

    def tail_vreg_body(k, kctr):
        ovec = oidx_v[pl.ds(k * _NL, _NL)]
        wv = owin_v[pl.ds(k * _NL, _NL)]
        m = wv == nwin
        cnt = jnp.sum(jnp.where(m, 1, 0))

        def entry_body(j, kctr):
            sel = lane == j
            idx_e = jnp.sum(jnp.where(sel, tmpi_v[...], 0))
            pos_e = jnp.sum(jnp.where(sel, tmpp_v[...], 0))

            @pl.when(kctr >= _RING)
            def _():
                pltpu.make_async_copy(
                    tail_hbm.at[0], rows_v.at[0], rsem
                ).wait()

            slot_r = kctr & (_RING - 1)
            rv = jnp.broadcast_to(idx_e - _FULL, (_NL,))
            for g in range(DIM // _NL):
                rows_v[slot_r, pl.ds(g * _NL, _NL)] = plsc.load_gather(
                    tail_v, [rv, lane + g * _NL]
                )
            pltpu.async_copy(
                rows_v.at[slot_r],
                out_hbm.at[pos_e >> 3, pos_e & 7],
                rsem,
            )
            return kctr + 1

        @pl.when(cnt > 0)
        def _():
            plsc.store_compressed(tmpi_v.at[pl.ds(0, _NL)], ovec, mask=m)
            plsc.store_compressed(
                tmpp_v.at[pl.ds(0, _NL)], opos_v[pl.ds(k * _NL, _NL)], mask=m
            )

        return lax.fori_loop(0, cnt, entry_body, kctr)

    kctr = lax.fori_loop(0, nvreg, tail_vreg_body, kctr)

    def drain_body(d, carry):
        pltpu.make_async_copy(tail_hbm.at[0], rows_v.at[0], rsem).wait()
        return carry

    lax.fori_loop(0, jnp.minimum(kctr, _RING), drain_body, 0)


def kernel(sample_ids, codebook):
    idx = sample_ids.astype(jnp.int32)
    tail = codebook[_FULL:]
    out3 = _gather_kernel(idx, codebook.T, tail)
    return out3.reshape(BATCH, DIM)
